# baseline (speedup 1.0000x reference)
import functools

import jax
import jax.numpy as jnp
from jax import lax
from jax.experimental import pallas as pl
from jax.experimental.pallas import tpu as pltpu
from jax.experimental.pallas import tpu_sc as plsc

_info = plsc.get_sparse_core_info()
_NC, _NS = _info.num_cores, _info.num_subcores
_NW = _NC * _NS

_CHUNK = 1600


@functools.cache
def _make_gather(B, V, D):
    assert B % (_NW * _CHUNK) == 0
    b_per_w = B // _NW
    n_chunks = b_per_w // _CHUNK
    assert n_chunks % 2 == 0 and n_chunks >= 6
    mesh = plsc.VectorSubcoreMesh(core_axis_name="c", subcore_axis_name="s")

    @functools.partial(
        pl.kernel,
        mesh=mesh,
        out_type=jax.ShapeDtypeStruct((B, D), jnp.bfloat16),
        scratch_types=[
            pltpu.VMEM_SHARED((V, D), jnp.bfloat16),
            pltpu.VMEM((_CHUNK,), jnp.int32),
            pltpu.VMEM((_CHUNK,), jnp.int32),
            pltpu.VMEM((_CHUNK, D), jnp.bfloat16),
            pltpu.VMEM((_CHUNK, D), jnp.bfloat16),
            pltpu.SemaphoreType.DMA,
            pltpu.SemaphoreType.DMA,
            pltpu.SemaphoreType.DMA,
            pltpu.SemaphoreType.DMA,
            pltpu.SemaphoreType.DMA,
        ],
        compiler_params=pltpu.CompilerParams(use_tc_tiling_on_sc=False,
                                             needs_layout_passes=False),
    )
    def gather_kernel(table_hbm, idx_hbm, out_hbm, table_s, idx0, idx1,
                      rows0, rows1, sem_i0, sem_i1, sem_g, sem_o0, sem_o1):
        sid = lax.axis_index("s")
        wid = sid * _NC + lax.axis_index("c")
        base = wid * b_per_w
        idx_v = (idx0, idx1)
        rows_v = (rows0, rows1)
        sem_i = (sem_i0, sem_i1)
        sem_o = (sem_o0, sem_o1)

        @pl.when(sid == 0)
        def _stage():
            pltpu.sync_copy(table_hbm, table_s)

        plsc.subcore_barrier()

        def issue_idx(g, b):
            pltpu.async_copy(idx_hbm.at[pl.ds(base + g * _CHUNK, _CHUNK)],
                             idx_v[b], sem_i[b])

        def wait_idx(b):
            pltpu.make_async_copy(idx_hbm.at[pl.ds(0, _CHUNK)], idx_v[b],
                                  sem_i[b]).wait()

        def issue_out(g, b):
            pltpu.async_copy(rows_v[b],
                             out_hbm.at[pl.ds(base + g * _CHUNK, _CHUNK)],
                             sem_o[b])

        def wait_out(b):
            pltpu.make_async_copy(rows_v[b], out_hbm.at[pl.ds(0, _CHUNK)],
                                  sem_o[b]).wait()

        def body(g, b, first, last):
            if not first:
                wait_out(b)
            wait_idx(b)
            pltpu.async_copy(table_s.at[idx_v[b]], rows_v[b], sem_g).wait()
            if not last:
                issue_idx(g + 2, b)
            issue_out(g, b)

        issue_idx(0, 0)
        issue_idx(1, 1)
        body(0, 0, first=True, last=False)
        body(1, 1, first=True, last=False)

        def outer(o, _):
            g = 2 * o
            body(g, 0, first=False, last=False)
            body(g + 1, 1, first=False, last=False)
            return ()

        lax.fori_loop(1, n_chunks // 2 - 1, outer, ())

        body(n_chunks - 2, 0, first=False, last=True)
        body(n_chunks - 1, 1, first=False, last=True)
        wait_out(0)
        wait_out(1)

    return gather_kernel


def kernel(word_sequences, embedding_table):
    Bo, T = word_sequences.shape
    V, D = embedding_table.shape
    flat_idx = word_sequences.reshape(-1)
    table_bf16 = embedding_table.astype(jnp.bfloat16)
    out = _make_gather(Bo * T, V, D)(table_bf16, flat_idx)
    return out.reshape(Bo, T, D).astype(jnp.float32)


# speedup vs baseline: 1.5732x; 1.5732x over previous
import functools

import jax
import jax.numpy as jnp
from jax import lax
from jax.experimental import pallas as pl
from jax.experimental.pallas import tpu as pltpu
from jax.experimental.pallas import tpu_sc as plsc

_info = plsc.get_sparse_core_info()
_NC, _NS = _info.num_cores, _info.num_subcores
_NW = _NC * _NS

_CHUNK = 800
_NBUF = 4


@functools.cache
def _make_gather(B, V, D):
    assert B % (_NW * _CHUNK) == 0
    b_per_w = B // _NW
    n_chunks = b_per_w // _CHUNK
    assert n_chunks % _NBUF == 0 and n_chunks >= 3 * _NBUF
    mesh = plsc.VectorSubcoreMesh(core_axis_name="c", subcore_axis_name="s")

    @functools.partial(
        pl.kernel,
        mesh=mesh,
        out_type=jax.ShapeDtypeStruct((B, D), jnp.float32),
        scratch_types=[
            pltpu.VMEM_SHARED((V, D), jnp.float32),
            [pltpu.VMEM((_CHUNK,), jnp.int32)] * _NBUF,
            [pltpu.VMEM((_CHUNK, D), jnp.float32)] * _NBUF,
            [pltpu.SemaphoreType.DMA] * _NBUF,
            pltpu.SemaphoreType.DMA,
            [pltpu.SemaphoreType.DMA] * _NBUF,
        ],
        compiler_params=pltpu.CompilerParams(use_tc_tiling_on_sc=False,
                                             needs_layout_passes=False),
    )
    def gather_kernel(table_hbm, idx_hbm, out_hbm, table_s, idx_v, rows_v,
                      sem_i, sem_g, sem_o):
        sid = lax.axis_index("s")
        wid = sid * _NC + lax.axis_index("c")
        base = wid * b_per_w

        @pl.when(sid == 0)
        def _stage():
            pltpu.sync_copy(table_hbm, table_s)

        plsc.subcore_barrier()

        def issue_idx(g, b):
            pltpu.async_copy(idx_hbm.at[pl.ds(base + g * _CHUNK, _CHUNK)],
                             idx_v[b], sem_i[b])

        def wait_idx(b):
            pltpu.make_async_copy(idx_hbm.at[pl.ds(0, _CHUNK)], idx_v[b],
                                  sem_i[b]).wait()

        def issue_out(g, b):
            pltpu.async_copy(rows_v[b],
                             out_hbm.at[pl.ds(base + g * _CHUNK, _CHUNK)],
                             sem_o[b])

        def wait_out(b):
            pltpu.make_async_copy(rows_v[b], out_hbm.at[pl.ds(0, _CHUNK)],
                                  sem_o[b]).wait()

        def body(g, b, first, last):
            if not first:
                wait_out(b)
            wait_idx(b)
            pltpu.async_copy(table_s.at[idx_v[b]], rows_v[b], sem_g).wait()
            if not last:
                issue_idx(g + _NBUF, b)
            issue_out(g, b)

        for b in range(_NBUF):
            issue_idx(b, b)
        for b in range(_NBUF):
            body(b, b, first=True, last=False)

        def outer(o, _):
            g0 = o * _NBUF
            for b in range(_NBUF):
                body(g0 + b, b, first=False, last=False)
            return ()

        lax.fori_loop(1, n_chunks // _NBUF - 1, outer, ())

        for b in range(_NBUF):
            body(n_chunks - _NBUF + b, b, first=False, last=True)
        for b in range(_NBUF):
            wait_out(b)

    return gather_kernel


def kernel(word_sequences, embedding_table):
    Bo, T = word_sequences.shape
    V, D = embedding_table.shape
    flat_idx = word_sequences.reshape(-1)
    out = _make_gather(Bo * T, V, D)(embedding_table, flat_idx)
    return out.reshape(Bo, T, D)
